# baseline (speedup 1.0000x reference)
import functools

import jax
import jax.numpy as jnp
from jax import lax
from jax.experimental import pallas as pl
from jax.experimental.pallas import tpu as pltpu
from jax.experimental.pallas import tpu_sc as plsc

N_EMBED = 1_000_000
D_MODEL = 64
S_TOTAL = 16384
T_PLANES = 50
NC, NS = 2, 16
NW = NC * NS
CH = 128
SBLK = S_TOTAL // CH
NCHUNK = T_PLANES * SBLK // NW
NBUF = 4
L = 16

_MESH = plsc.VectorSubcoreMesh(
    core_axis_name="c", subcore_axis_name="s", num_cores=NC, num_subcores=NS
)


@functools.partial(
    pl.kernel,
    out_type=jax.ShapeDtypeStruct((T_PLANES, D_MODEL, S_TOTAL), jnp.float32),
    mesh=_MESH,
    scratch_types=[
        pltpu.VMEM((NCHUNK, CH), jnp.int32),
        pltpu.VMEM((NBUF, CH, D_MODEL), jnp.float32),
        pltpu.VMEM((2, D_MODEL, CH), jnp.float32),
    ]
    + [pltpu.SemaphoreType.DMA] * (NBUF + 2),
    compiler_params=pltpu.CompilerParams(
        use_tc_tiling_on_sc=False, needs_layout_passes=False
    ),
)
def _embed_gather(idx_hbm, table_hbm, out_hbm, idx_v, rows_v, tr_v, *sems):
    g_sems, s_sems = sems[:NBUF], sems[NBUF:]
    wid = lax.axis_index("s") * NC + lax.axis_index("c")
    cbase = wid * NCHUNK

    pltpu.sync_copy(idx_hbm.at[wid], idx_v)

    iota = lax.iota(jnp.int32, L)

    def fire_gather(j, b):
        pltpu.async_copy(table_hbm.at[idx_v.at[j]], rows_v.at[b], g_sems[b])

    def drain_gather(b):
        pltpu.make_async_copy(
            table_hbm.at[idx_v.at[0]], rows_v.at[b], g_sems[b]
        ).wait()

    def transpose(b, p):
        src = rows_v.at[b]
        dst = tr_v.at[p]

        @pl.loop(0, CH // L)
        def _(k):
            row_ids = iota + k * L
            for d in range(D_MODEL):
                vals = plsc.load_gather(
                    src, [row_ids, jnp.full((L,), d, jnp.int32)]
                )
                dst[d, pl.ds(k * L, L)] = vals

    def fire_store(j, p):
        c = cbase + j
        t = c // SBLK
        sb = c % SBLK
        pltpu.async_copy(
            tr_v.at[p], out_hbm.at[t, :, pl.ds(sb * CH, CH)], s_sems[p]
        )

    def drain_store(p):
        pltpu.make_async_copy(
            tr_v.at[p], out_hbm.at[0, :, pl.ds(0, CH)], s_sems[p]
        ).wait()

    def step(j, b, p, do_drain_store, do_fire):
        drain_gather(b)
        if do_drain_store:
            drain_store(p)
        transpose(b, p)
        fire_store(j, p)
        if do_fire:
            fire_gather(j + NBUF, b)

    for b in range(NBUF):
        fire_gather(b, b)
    for j in range(NBUF):
        step(j, j, j % 2, j >= 2, True)

    @pl.loop(NBUF, NCHUNK - NBUF, step=NBUF)
    def _(tj):
        for dj in range(NBUF):
            step(tj + dj, dj, dj % 2, True, True)

    for j in range(NCHUNK - NBUF, NCHUNK):
        step(j, j % NBUF, j % 2, True, False)
    for p in range(2):
        drain_store(p)


def kernel(x, weight):
    idx = x.T.astype(jnp.int32).reshape(NW, NCHUNK, CH)
    out = _embed_gather(idx, weight)
    return jnp.transpose(out, (2, 0, 1))


# speedup vs baseline: 1.6661x; 1.6661x over previous
import functools

import jax
import jax.numpy as jnp
from jax import lax
from jax.experimental import pallas as pl
from jax.experimental.pallas import tpu as pltpu
from jax.experimental.pallas import tpu_sc as plsc

N_EMBED = 1_000_000
D_MODEL = 64
S_TOTAL = 16384
T_PLANES = 50
NC, NS = 2, 16
NW = NC * NS
CH = 128
SBLK = S_TOTAL // CH
NCHUNK = T_PLANES * SBLK // NW
NBUF = 4
L = 16

_MESH = plsc.VectorSubcoreMesh(
    core_axis_name="c", subcore_axis_name="s", num_cores=NC, num_subcores=NS
)


@functools.partial(
    pl.kernel,
    out_type=jax.ShapeDtypeStruct((T_PLANES, D_MODEL, S_TOTAL), jnp.float32),
    mesh=_MESH,
    scratch_types=[
        pltpu.VMEM((NCHUNK, CH), jnp.int32),
        pltpu.VMEM((NBUF, CH, D_MODEL), jnp.float32),
        pltpu.VMEM((2, D_MODEL, CH), jnp.float32),
    ]
    + [pltpu.SemaphoreType.DMA] * (NBUF + 2),
    compiler_params=pltpu.CompilerParams(
        use_tc_tiling_on_sc=False, needs_layout_passes=False
    ),
)
def _embed_gather(idx_hbm, table_hbm, out_hbm, idx_v, rows_v, tr_v, *sems):
    g_sems, s_sems = sems[:NBUF], sems[NBUF:]
    wid = lax.axis_index("s") * NC + lax.axis_index("c")
    cbase = wid * NCHUNK

    pltpu.sync_copy(idx_hbm.at[wid], idx_v)

    iota = lax.iota(jnp.int32, L)

    def fire_gather(j, b):
        pltpu.async_copy(table_hbm.at[idx_v.at[j]], rows_v.at[b], g_sems[b])

    def drain_gather(b):
        pltpu.make_async_copy(
            table_hbm.at[idx_v.at[0]], rows_v.at[b], g_sems[b]
        ).wait()

    def transpose(b, p):
        src = rows_v.at[b]
        dst = tr_v.at[p]
        row_vecs = [iota + br for br in range(0, CH, L)]

        @pl.loop(0, L)
        def _(s):
            skew = (iota + s) & (L - 1)
            for br in range(CH // L):
                row_vec = row_vecs[br]
                for bd in range(0, D_MODEL, L):
                    col_vec = skew + bd
                    vals = plsc.load_gather(src, [row_vec, col_vec])
                    plsc.store_scatter(dst, [col_vec, row_vec], vals)

    def fire_store(j, p):
        c = cbase + j
        t = c // SBLK
        sb = c % SBLK
        pltpu.async_copy(
            tr_v.at[p], out_hbm.at[t, :, pl.ds(sb * CH, CH)], s_sems[p]
        )

    def drain_store(p):
        pltpu.make_async_copy(
            tr_v.at[p], out_hbm.at[0, :, pl.ds(0, CH)], s_sems[p]
        ).wait()

    def step(j, b, p, do_drain_store, do_fire):
        drain_gather(b)
        if do_drain_store:
            drain_store(p)
        transpose(b, p)
        fire_store(j, p)
        if do_fire:
            fire_gather(j + NBUF, b)

    for b in range(NBUF):
        fire_gather(b, b)
    for j in range(NBUF):
        step(j, j, j % 2, j >= 2, True)

    @pl.loop(NBUF, NCHUNK - NBUF, step=NBUF)
    def _(tj):
        for dj in range(NBUF):
            step(tj + dj, dj, dj % 2, True, True)

    for j in range(NCHUNK - NBUF, NCHUNK):
        step(j, j % NBUF, j % 2, True, False)
    for p in range(2):
        drain_store(p)


def kernel(x, weight):
    idx = x.T.astype(jnp.int32).reshape(NW, NCHUNK, CH)
    out = _embed_gather(idx, weight)
    return jnp.transpose(out, (2, 0, 1))


# speedup vs baseline: 1.9852x; 1.1915x over previous
import functools

import jax
import jax.numpy as jnp
from jax import lax
from jax.experimental import pallas as pl
from jax.experimental.pallas import tpu as pltpu
from jax.experimental.pallas import tpu_sc as plsc

N_EMBED = 1_000_000
D_MODEL = 64
S_TOTAL = 16384
T_PLANES = 50
NC, NS = 2, 16
NW = NC * NS
CH = 128
SBLK = S_TOTAL // CH
NCHUNK = T_PLANES * SBLK // NW
NBUF = 4
L = 16

_MESH = plsc.VectorSubcoreMesh(
    core_axis_name="c", subcore_axis_name="s", num_cores=NC, num_subcores=NS
)


@functools.partial(
    pl.kernel,
    out_type=jax.ShapeDtypeStruct((T_PLANES, D_MODEL, S_TOTAL), jnp.float32),
    mesh=_MESH,
    scratch_types=[
        pltpu.VMEM((NCHUNK, CH), jnp.int32),
        pltpu.VMEM((NBUF, CH, D_MODEL), jnp.float32),
        pltpu.VMEM((2, D_MODEL, CH), jnp.float32),
    ]
    + [pltpu.SemaphoreType.DMA] * (NBUF + 2),
    compiler_params=pltpu.CompilerParams(
        use_tc_tiling_on_sc=False, needs_layout_passes=False
    ),
)
def _embed_gather(idx_hbm, table_hbm, out_hbm, idx_v, rows_v, tr_v, *sems):
    g_sems, s_sems = sems[:NBUF], sems[NBUF:]
    wid = lax.axis_index("s") * NC + lax.axis_index("c")
    cbase = wid * NCHUNK

    pltpu.sync_copy(idx_hbm.at[wid], idx_v)

    iota = lax.iota(jnp.int32, L)

    def fire_gather(j, b):
        pltpu.async_copy(table_hbm.at[idx_v.at[j]], rows_v.at[b], g_sems[b])

    def drain_gather(b):
        pltpu.make_async_copy(
            table_hbm.at[idx_v.at[0]], rows_v.at[b], g_sems[b]
        ).wait()

    def transpose(b, p):
        src = rows_v.at[b]
        dst = tr_v.at[p]
        row_vecs = [iota + br for br in range(0, CH, L)]

        @plsc.parallel_loop(0, L)
        def _(s):
            skew = (iota + s) & (L - 1)
            for br in range(CH // L):
                row_vec = row_vecs[br]
                for bd in range(0, D_MODEL, L):
                    col_vec = skew + bd
                    vals = plsc.load_gather(src, [row_vec, col_vec])
                    plsc.store_scatter(dst, [col_vec, row_vec], vals)

    def fire_store(j, p):
        c = cbase + j
        t = c // SBLK
        sb = c % SBLK
        pltpu.async_copy(
            tr_v.at[p], out_hbm.at[t, :, pl.ds(sb * CH, CH)], s_sems[p]
        )

    def drain_store(p):
        pltpu.make_async_copy(
            tr_v.at[p], out_hbm.at[0, :, pl.ds(0, CH)], s_sems[p]
        ).wait()

    def step(j, b, p, do_drain_store, do_fire):
        drain_gather(b)
        if do_drain_store:
            drain_store(p)
        transpose(b, p)
        fire_store(j, p)
        if do_fire:
            fire_gather(j + NBUF, b)

    for b in range(NBUF):
        fire_gather(b, b)
    for j in range(NBUF):
        step(j, j, j % 2, j >= 2, True)

    @pl.loop(NBUF, NCHUNK - NBUF, step=NBUF)
    def _(tj):
        for dj in range(NBUF):
            step(tj + dj, dj, dj % 2, True, True)

    for j in range(NCHUNK - NBUF, NCHUNK):
        step(j, j % NBUF, j % 2, True, False)
    for p in range(2):
        drain_store(p)


def kernel(x, weight):
    idx = x.T.astype(jnp.int32).reshape(NW, NCHUNK, CH)
    out = _embed_gather(idx, weight)
    return jnp.transpose(out, (2, 0, 1))


# speedup vs baseline: 3.7365x; 1.8822x over previous
import functools

import jax
import jax.numpy as jnp
from jax import lax
from jax.experimental import pallas as pl
from jax.experimental.pallas import tpu as pltpu
from jax.experimental.pallas import tpu_sc as plsc

N_EMBED = 1_000_000
D_MODEL = 64
S_TOTAL = 16384
T_PLANES = 50
NC, NS = 2, 16
NW = NC * NS
CH = 128
SBLK = S_TOTAL // CH
NCHUNK = T_PLANES * SBLK // NW
NBUF = 4
L = 16

NTILE = (N_EMBED + 127) // 128
KFULL = 244
W2_ROWS = N_EMBED // 2

_MESH = plsc.VectorSubcoreMesh(
    core_axis_name="c", subcore_axis_name="s", num_cores=NC, num_subcores=NS
)
_PARAMS = pltpu.CompilerParams(
    use_tc_tiling_on_sc=True, needs_layout_passes=False
)


@functools.partial(
    pl.kernel,
    out_type=jax.ShapeDtypeStruct((W2_ROWS, 128), jnp.float32),
    mesh=_MESH,
    scratch_types=[
        pltpu.VMEM((2, D_MODEL, 128), jnp.float32),
        pltpu.VMEM((2, D_MODEL, 128), jnp.float32),
    ]
    + [pltpu.SemaphoreType.DMA] * 4,
    compiler_params=_PARAMS,
)
def _retile(wt_hbm, wtail_hbm, w2_hbm, in_v, tt_v, *sems):
    in_sems, out_sems = sems[:2], sems[2:]
    wid = lax.axis_index("s") * NC + lax.axis_index("c")
    iota = lax.iota(jnp.int32, L)
    j16 = [iota + bj * L for bj in range(8)]
    q_vecs = [v >> 1 for v in j16]
    h64_vecs = [(v & 1) << 6 for v in j16]

    def fire_in(c, b):
        pltpu.async_copy(
            wt_hbm.at[:, pl.ds(c * 128, 128)], in_v.at[b], in_sems[b]
        )

    def drain_in(b):
        pltpu.make_async_copy(
            wt_hbm.at[:, pl.ds(0, 128)], in_v.at[b], in_sems[b]
        ).wait()

    def fire_out(c, b):
        pltpu.async_copy(
            tt_v.at[b], w2_hbm.at[pl.ds(c * 64, 64), :], out_sems[b]
        )

    def drain_out(b):
        pltpu.make_async_copy(
            tt_v.at[b], w2_hbm.at[pl.ds(0, 64), :], out_sems[b]
        ).wait()

    def transpose(b, nbj):
        src = in_v.at[b]
        dst = tt_v.at[b]

        @plsc.parallel_loop(0, L)
        def _(s):
            skew = (iota + s) & (L - 1)
            for bj in range(nbj):
                for bd in range(0, D_MODEL, L):
                    dvec = skew + bd
                    vals = plsc.load_gather(src, [dvec, j16[bj]])
                    plsc.store_scatter(dst, [q_vecs[bj], h64_vecs[bj] + dvec], vals)

    def step(kk, b, do_drain_out, do_fire_in):
        c = wid + kk * NW
        drain_in(b)
        if do_drain_out:
            drain_out(b)
        transpose(b, 8)
        fire_out(c, b)
        if do_fire_in:
            fire_in(wid + (kk + 2) * NW, b)

    fire_in(wid, 0)
    fire_in(wid + NW, 1)
    for kk in range(2):
        step(kk, kk, False, True)

    @pl.loop(2, KFULL - 2, step=2)
    def _(k):
        for b in range(2):
            step(k + b, b, True, True)

    for kk in range(KFULL - 2, KFULL):
        step(kk, kk % 2, True, False)
    for b in range(2):
        drain_out(b)

    @pl.when(wid < 4)
    def _():
        c = KFULL * NW + wid
        pltpu.sync_copy(wt_hbm.at[:, pl.ds(c * 128, 128)], in_v.at[0])
        transpose(0, 8)
        pltpu.sync_copy(tt_v.at[0], w2_hbm.at[pl.ds(c * 64, 64), :])

    @pl.when(wid == 4)
    def _():
        pltpu.sync_copy(wtail_hbm, in_v.at[0])
        transpose(0, 8)
        pltpu.sync_copy(
            tt_v.at[0].at[pl.ds(32, 32), :],
            w2_hbm.at[pl.ds(W2_ROWS - 32, 32), :],
        )


@functools.partial(
    pl.kernel,
    out_type=jax.ShapeDtypeStruct((T_PLANES, D_MODEL, S_TOTAL), jnp.float32),
    mesh=_MESH,
    scratch_types=[
        pltpu.VMEM((NCHUNK, CH), jnp.int32),
        pltpu.VMEM((NBUF, CH), jnp.int32),
        pltpu.VMEM((NBUF, CH, 128), jnp.float32),
        pltpu.VMEM((2, D_MODEL, CH), jnp.float32),
    ]
    + [pltpu.SemaphoreType.DMA] * (NBUF + 2),
    compiler_params=_PARAMS,
)
def _gather2(idx_hbm, w2_hbm, out_hbm, idx_v, idx_s, rows_v, tr_v, *sems):
    g_sems, s_sems = sems[:NBUF], sems[NBUF:]
    wid = lax.axis_index("s") * NC + lax.axis_index("c")
    cbase = wid * NCHUNK

    pltpu.sync_copy(idx_hbm.at[wid], idx_v)

    iota = lax.iota(jnp.int32, L)
    row_vecs = [iota + br for br in range(0, CH, L)]

    def stage(j, b):
        for r in range(0, CH, L):
            idx_s[b, pl.ds(r, L)] = idx_v.at[j][pl.ds(r, L)] >> 1

    def fire_gather(j, b):
        stage(j, b)
        pltpu.async_copy(w2_hbm.at[idx_s.at[b]], rows_v.at[b], g_sems[b])

    def drain_gather(b):
        pltpu.make_async_copy(
            w2_hbm.at[idx_s.at[b]], rows_v.at[b], g_sems[b]
        ).wait()

    def transpose(j, b, p):
        src = rows_v.at[b]
        dst = tr_v.at[p]
        par64 = [
            (idx_v.at[j][pl.ds(br, L)] & 1) << 6 for br in range(0, CH, L)
        ]

        @plsc.parallel_loop(0, L)
        def _(s):
            skew = (iota + s) & (L - 1)
            for br in range(CH // L):
                for bd in range(0, D_MODEL, L):
                    dvec = skew + bd
                    vals = plsc.load_gather(
                        src, [row_vecs[br], par64[br] + dvec]
                    )
                    plsc.store_scatter(dst, [dvec, row_vecs[br]], vals)

    def fire_store(j, p):
        c = cbase + j
        t = c // SBLK
        sb = c % SBLK
        pltpu.async_copy(
            tr_v.at[p], out_hbm.at[t, :, pl.ds(sb * CH, CH)], s_sems[p]
        )

    def drain_store(p):
        pltpu.make_async_copy(
            tr_v.at[p], out_hbm.at[0, :, pl.ds(0, CH)], s_sems[p]
        ).wait()

    def step(j, b, p, do_drain_store, do_fire):
        drain_gather(b)
        if do_drain_store:
            drain_store(p)
        transpose(j, b, p)
        fire_store(j, p)
        if do_fire:
            fire_gather(j + NBUF, b)

    for b in range(NBUF):
        fire_gather(b, b)
    for j in range(NBUF):
        step(j, j, j % 2, j >= 2, True)

    @pl.loop(NBUF, NCHUNK - NBUF, step=NBUF)
    def _(tj):
        for dj in range(NBUF):
            step(tj + dj, dj, dj % 2, True, True)

    for j in range(NCHUNK - NBUF, NCHUNK):
        step(j, j % NBUF, j % 2, True, False)
    for p in range(2):
        drain_store(p)


def kernel(x, weight):
    idx = x.T.astype(jnp.int32).reshape(NW, NCHUNK, CH)
    wtail = weight[N_EMBED - 128 :].T
    w2 = _retile(weight.T, wtail)
    out = _gather2(idx, w2)
    return jnp.transpose(out, (2, 0, 1))
